# baseline (speedup 1.0000x reference)
import functools

import jax
import jax.numpy as jnp
from jax import lax
from jax.experimental import pallas as pl
from jax.experimental.pallas import tpu as pltpu
from jax.experimental.pallas import tpu_sc as plsc

_GROUP = 32
_NBUF = 8
_CHUNKS = 20
_NSUB = 16
_NCORE = 2
_LANES = 16


def _linear_split_kernel(x_ref, w_ref, b_ref, o_ref):
    res = lax.dot_general(
        x_ref[...], w_ref[...], (((1,), (1,)), ((), ())),
        preferred_element_type=jnp.float32,
        precision=lax.Precision.HIGHEST) + b_ref[...]
    h = res.shape[1] // 2
    o_ref[0] = res[:, :h]
    o_ref[1] = res[:, h:]


def _combine_kernel(x_ref, a_ref, ws_ref, bs_ref, wc_ref, bc_ref, o_ref):
    aggr = jnp.concatenate([a_ref[0], a_ref[1]], axis=-1)
    t = lax.dot_general(
        x_ref[...], ws_ref[...], (((1,), (1,)), ((), ())),
        preferred_element_type=jnp.float32,
        precision=lax.Precision.HIGHEST) + bs_ref[...] + aggr
    o_ref[...] = lax.dot_general(
        t, wc_ref[...], (((1,), (1,)), ((), ())),
        preferred_element_type=jnp.float32,
        precision=lax.Precision.HIGHEST) + bc_ref[...]


def _sc_segment_sum(hsplit, src2, dst2, n_nodes, groups_per_sub, acc_rows):
    h = hsplit.shape[2]
    mesh = plsc.VectorSubcoreMesh(core_axis_name="c", subcore_axis_name="s")
    zper = acc_rows // _NSUB
    half = groups_per_sub // _CHUNKS

    @functools.partial(
        pl.kernel,
        out_type=jax.ShapeDtypeStruct((_NCORE, acc_rows, h), jnp.float32),
        mesh=mesh,
        scratch_types=(
            [pltpu.VMEM((half, _GROUP), jnp.int32)] * 4
            + [pltpu.VMEM((_GROUP, h), jnp.float32)] * _NBUF
            + [pltpu.VMEM_SHARED((acc_rows, h), jnp.float32)]
            + [pltpu.SemaphoreType.DMA] * (_NBUF + 2)
        ),
    )
    def k(h_hbm, s_hbm, d_hbm, o_hbm, *rest):
        slabs = ((rest[0], rest[1]), (rest[2], rest[3]))
        bufs = rest[4:4 + _NBUF]
        acc = rest[4 + _NBUF]
        sems = rest[5 + _NBUF:5 + 2 * _NBUF]
        lsem0, lsem1 = rest[5 + 2 * _NBUF], rest[6 + 2 * _NBUF]
        c = lax.axis_index("c")
        s = lax.axis_index("s")
        rows = bufs[0]
        hc = h_hbm.at[c]

        @pl.loop(0, _GROUP)
        def _(r):
            @pl.loop(0, h, step=_LANES)
            def _(i):
                rows.at[pl.ds(r, 1), pl.ds(i, _LANES)][...] = (
                    jnp.zeros((1, _LANES), jnp.float32))

        zfull = (zper // _GROUP) * _GROUP

        @pl.loop(0, zfull, step=_GROUP)
        def _(r0):
            pltpu.sync_copy(rows, acc.at[pl.ds(s * zper + r0, _GROUP)])
        if zfull != zper:
            pltpu.sync_copy(rows.at[pl.ds(0, zper - zfull)],
                            acc.at[pl.ds(s * zper + zfull, zper - zfull)])

        plsc.subcore_barrier()

        base = s * groups_per_sub
        pltpu.sync_copy(s_hbm.at[pl.ds(base, half)], slabs[0][0])
        pltpu.sync_copy(d_hbm.at[pl.ds(base, half)], slabs[0][1])

        for b in range(_NBUF):
            pltpu.async_copy(hc.at[slabs[0][0].at[b]], bufs[b], sems[b])

        for ch in range(_CHUNKS):
            sA, dA = slabs[ch % 2]
            last = ch + 1 == _CHUNKS
            if not last:
                sB, dB = slabs[(ch + 1) % 2]
                nb = base + (ch + 1) * half
                pltpu.async_copy(s_hbm.at[pl.ds(nb, half)], sB, lsem0)
                pltpu.async_copy(d_hbm.at[pl.ds(nb, half)], dB, lsem1)

            @pl.loop(0, half - _NBUF, step=_NBUF)
            def _(j, sA=sA, dA=dA):
                for b in range(_NBUF):
                    pltpu.make_async_copy(hc.at[sA.at[j + b]], bufs[b],
                                          sems[b]).wait()
                    pltpu.sync_copy(bufs[b], acc.at[dA.at[j + b]], add=True)
                    pltpu.async_copy(hc.at[sA.at[j + b + _NBUF]], bufs[b],
                                     sems[b])

            if not last:
                pltpu.make_async_copy(s_hbm.at[pl.ds(nb, half)], sB,
                                      lsem0).wait()
                pltpu.make_async_copy(d_hbm.at[pl.ds(nb, half)], dB,
                                      lsem1).wait()

            for b in range(_NBUF):
                pltpu.make_async_copy(hc.at[sA.at[half - _NBUF + b]], bufs[b],
                                      sems[b]).wait()
                pltpu.sync_copy(bufs[b], acc.at[dA.at[half - _NBUF + b]],
                                add=True)
                if not last:
                    pltpu.async_copy(hc.at[sB.at[b]], bufs[b], sems[b])

        plsc.subcore_barrier()

        pltpu.sync_copy(acc.at[pl.ds(s * zper, zper)],
                        o_hbm.at[c, pl.ds(s * zper, zper)])

    return k(hsplit, src2, dst2)


def kernel(feat, edge_index, W_aggr, b_aggr, W_self, b_self, W_comb, b_comb):
    n, d = feat.shape
    e = edge_index.shape[1]
    h = d // 2
    m_blk = 1000
    grid = n // m_blk

    hsplit = pl.pallas_call(
        _linear_split_kernel,
        grid=(grid,),
        in_specs=[pl.BlockSpec((m_blk, d), lambda i: (i, 0)),
                  pl.BlockSpec((d, d), lambda i: (0, 0)),
                  pl.BlockSpec((1, d), lambda i: (0, 0))],
        out_specs=pl.BlockSpec((2, m_blk, h), lambda i: (0, i, 0)),
        out_shape=jax.ShapeDtypeStruct((2, n, h), jnp.float32),
    )(feat, W_aggr, b_aggr.reshape(1, d))

    unit = _GROUP * _NSUB * 8
    e_pad = ((e + unit - 1) // unit) * unit
    src = edge_index[0].astype(jnp.int32)
    dst = edge_index[1].astype(jnp.int32)
    if e_pad != e:
        src = jnp.concatenate([src, jnp.zeros((e_pad - e,), jnp.int32)])
        dst = jnp.concatenate([dst, jnp.full((e_pad - e,), n, jnp.int32)])
    src2 = src.reshape(e_pad // _GROUP, _GROUP)
    dst2 = dst.reshape(e_pad // _GROUP, _GROUP)
    groups_per_sub = e_pad // (_GROUP * _NSUB)

    acc_rows = ((n + (1 if e_pad != e else 0) + 127) // 128) * 128

    aggr = _sc_segment_sum(hsplit, src2, dst2, n, groups_per_sub, acc_rows)

    out = pl.pallas_call(
        _combine_kernel,
        grid=(grid,),
        in_specs=[pl.BlockSpec((m_blk, d), lambda i: (i, 0)),
                  pl.BlockSpec((2, m_blk, h), lambda i: (0, i, 0)),
                  pl.BlockSpec((d, d), lambda i: (0, 0)),
                  pl.BlockSpec((1, d), lambda i: (0, 0)),
                  pl.BlockSpec((d, d), lambda i: (0, 0)),
                  pl.BlockSpec((1, d), lambda i: (0, 0))],
        out_specs=pl.BlockSpec((m_blk, d), lambda i: (i, 0)),
        out_shape=jax.ShapeDtypeStruct((n, d), jnp.float32),
    )(feat, aggr, W_self, b_self.reshape(1, d), W_comb, b_comb.reshape(1, d))
    return out


# speedup vs baseline: 1.0112x; 1.0112x over previous
import functools

import jax
import jax.numpy as jnp
from jax import lax
from jax.experimental import pallas as pl
from jax.experimental.pallas import tpu as pltpu
from jax.experimental.pallas import tpu_sc as plsc

_GROUP = 64
_NBUF = 4
_CHUNKS = 5
_NSUB = 16
_NCORE = 2
_LANES = 16


def _linear_split_kernel(x_ref, w_ref, b_ref, o_ref):
    res = lax.dot_general(
        x_ref[...], w_ref[...], (((1,), (1,)), ((), ())),
        preferred_element_type=jnp.float32,
        precision=lax.Precision.HIGHEST) + b_ref[...]
    h = res.shape[1] // 2
    o_ref[0] = res[:, :h]
    o_ref[1] = res[:, h:]


def _combine_kernel(x_ref, a_ref, ws_ref, bs_ref, wc_ref, bc_ref, o_ref):
    aggr = jnp.concatenate([a_ref[0], a_ref[1]], axis=-1)
    t = lax.dot_general(
        x_ref[...], ws_ref[...], (((1,), (1,)), ((), ())),
        preferred_element_type=jnp.float32,
        precision=lax.Precision.HIGHEST) + bs_ref[...] + aggr
    o_ref[...] = lax.dot_general(
        t, wc_ref[...], (((1,), (1,)), ((), ())),
        preferred_element_type=jnp.float32,
        precision=lax.Precision.HIGHEST) + bc_ref[...]


def _sc_segment_sum(hsplit, src2, dst2, n_nodes, groups_per_sub, acc_rows):
    h = hsplit.shape[2]
    mesh = plsc.VectorSubcoreMesh(core_axis_name="c", subcore_axis_name="s")
    zper = acc_rows // _NSUB
    half = groups_per_sub // _CHUNKS

    @functools.partial(
        pl.kernel,
        out_type=jax.ShapeDtypeStruct((_NCORE, acc_rows, h), jnp.float32),
        mesh=mesh,
        scratch_types=(
            [pltpu.VMEM((half, _GROUP), jnp.int32)] * 4
            + [pltpu.VMEM((_GROUP, h), jnp.float32)] * _NBUF
            + [pltpu.VMEM_SHARED((acc_rows, h), jnp.float32)]
            + [pltpu.SemaphoreType.DMA] * (_NBUF + 2)
        ),
    )
    def k(h_hbm, s_hbm, d_hbm, o_hbm, *rest):
        slabs = ((rest[0], rest[1]), (rest[2], rest[3]))
        bufs = rest[4:4 + _NBUF]
        acc = rest[4 + _NBUF]
        sems = rest[5 + _NBUF:5 + 2 * _NBUF]
        lsem0, lsem1 = rest[5 + 2 * _NBUF], rest[6 + 2 * _NBUF]
        c = lax.axis_index("c")
        s = lax.axis_index("s")
        rows = bufs[0]
        hc = h_hbm.at[c]

        @pl.loop(0, _GROUP)
        def _(r):
            @pl.loop(0, h, step=_LANES)
            def _(i):
                rows.at[pl.ds(r, 1), pl.ds(i, _LANES)][...] = (
                    jnp.zeros((1, _LANES), jnp.float32))

        zfull = (zper // _GROUP) * _GROUP

        @pl.loop(0, zfull, step=_GROUP)
        def _(r0):
            pltpu.sync_copy(rows, acc.at[pl.ds(s * zper + r0, _GROUP)])
        if zfull != zper:
            pltpu.sync_copy(rows.at[pl.ds(0, zper - zfull)],
                            acc.at[pl.ds(s * zper + zfull, zper - zfull)])

        plsc.subcore_barrier()

        base = s * groups_per_sub
        pltpu.sync_copy(s_hbm.at[pl.ds(base, half)], slabs[0][0])
        pltpu.sync_copy(d_hbm.at[pl.ds(base, half)], slabs[0][1])

        for b in range(_NBUF):
            pltpu.async_copy(hc.at[slabs[0][0].at[b]], bufs[b], sems[b])

        for ch in range(_CHUNKS):
            sA, dA = slabs[ch % 2]
            last = ch + 1 == _CHUNKS
            if not last:
                sB, dB = slabs[(ch + 1) % 2]
                nb = base + (ch + 1) * half
                pltpu.async_copy(s_hbm.at[pl.ds(nb, half)], sB, lsem0)
                pltpu.async_copy(d_hbm.at[pl.ds(nb, half)], dB, lsem1)

            @pl.loop(0, half - _NBUF, step=_NBUF)
            def _(j, sA=sA, dA=dA):
                for b in range(_NBUF):
                    pltpu.make_async_copy(hc.at[sA.at[j + b]], bufs[b],
                                          sems[b]).wait()
                    pltpu.sync_copy(bufs[b], acc.at[dA.at[j + b]], add=True)
                    pltpu.async_copy(hc.at[sA.at[j + b + _NBUF]], bufs[b],
                                     sems[b])

            if not last:
                pltpu.make_async_copy(s_hbm.at[pl.ds(nb, half)], sB,
                                      lsem0).wait()
                pltpu.make_async_copy(d_hbm.at[pl.ds(nb, half)], dB,
                                      lsem1).wait()

            for b in range(_NBUF):
                pltpu.make_async_copy(hc.at[sA.at[half - _NBUF + b]], bufs[b],
                                      sems[b]).wait()
                pltpu.sync_copy(bufs[b], acc.at[dA.at[half - _NBUF + b]],
                                add=True)
                if not last:
                    pltpu.async_copy(hc.at[sB.at[b]], bufs[b], sems[b])

        plsc.subcore_barrier()

        pltpu.sync_copy(acc.at[pl.ds(s * zper, zper)],
                        o_hbm.at[c, pl.ds(s * zper, zper)])

    return k(hsplit, src2, dst2)


def kernel(feat, edge_index, W_aggr, b_aggr, W_self, b_self, W_comb, b_comb):
    n, d = feat.shape
    e = edge_index.shape[1]
    h = d // 2
    m_blk = 1000
    grid = n // m_blk

    hsplit = pl.pallas_call(
        _linear_split_kernel,
        grid=(grid,),
        in_specs=[pl.BlockSpec((m_blk, d), lambda i: (i, 0)),
                  pl.BlockSpec((d, d), lambda i: (0, 0)),
                  pl.BlockSpec((1, d), lambda i: (0, 0))],
        out_specs=pl.BlockSpec((2, m_blk, h), lambda i: (0, i, 0)),
        out_shape=jax.ShapeDtypeStruct((2, n, h), jnp.float32),
    )(feat, W_aggr, b_aggr.reshape(1, d))

    unit = _GROUP * _NSUB * 8
    e_pad = ((e + unit - 1) // unit) * unit
    src = edge_index[0].astype(jnp.int32)
    dst = edge_index[1].astype(jnp.int32)
    if e_pad != e:
        src = jnp.concatenate([src, jnp.zeros((e_pad - e,), jnp.int32)])
        dst = jnp.concatenate([dst, jnp.full((e_pad - e,), n, jnp.int32)])
    src2 = src.reshape(e_pad // _GROUP, _GROUP)
    dst2 = dst.reshape(e_pad // _GROUP, _GROUP)
    groups_per_sub = e_pad // (_GROUP * _NSUB)

    acc_rows = ((n + (1 if e_pad != e else 0) + 127) // 128) * 128

    aggr = _sc_segment_sum(hsplit, src2, dst2, n, groups_per_sub, acc_rows)

    out = pl.pallas_call(
        _combine_kernel,
        grid=(grid,),
        in_specs=[pl.BlockSpec((m_blk, d), lambda i: (i, 0)),
                  pl.BlockSpec((2, m_blk, h), lambda i: (0, i, 0)),
                  pl.BlockSpec((d, d), lambda i: (0, 0)),
                  pl.BlockSpec((1, d), lambda i: (0, 0)),
                  pl.BlockSpec((d, d), lambda i: (0, 0)),
                  pl.BlockSpec((1, d), lambda i: (0, 0))],
        out_specs=pl.BlockSpec((m_blk, d), lambda i: (i, 0)),
        out_shape=jax.ShapeDtypeStruct((n, d), jnp.float32),
    )(feat, aggr, W_self, b_self.reshape(1, d), W_comb, b_comb.reshape(1, d))
    return out


# speedup vs baseline: 1.1070x; 1.0947x over previous
import functools

import jax
import jax.numpy as jnp
from jax import lax
from jax.experimental import pallas as pl
from jax.experimental.pallas import tpu as pltpu
from jax.experimental.pallas import tpu_sc as plsc

_GROUP = 64
_NBUF = 4
_CHUNKS = 5
_NSUB = 16
_NCORE = 2
_LANES = 16


def _linear_split_kernel(x_ref, w_ref, b_ref, o_ref):
    res = lax.dot_general(
        x_ref[...], w_ref[...], (((1,), (1,)), ((), ())),
        preferred_element_type=jnp.float32,
        precision=lax.Precision.DEFAULT) + b_ref[...]
    h = res.shape[1] // 2
    o_ref[0] = res[:, :h]
    o_ref[1] = res[:, h:]


def _combine_kernel(x_ref, a_ref, ws_ref, bs_ref, wc_ref, bc_ref, o_ref):
    aggr = jnp.concatenate([a_ref[0], a_ref[1]], axis=-1)
    t = lax.dot_general(
        x_ref[...], ws_ref[...], (((1,), (1,)), ((), ())),
        preferred_element_type=jnp.float32,
        precision=lax.Precision.DEFAULT) + bs_ref[...] + aggr
    o_ref[...] = lax.dot_general(
        t, wc_ref[...], (((1,), (1,)), ((), ())),
        preferred_element_type=jnp.float32,
        precision=lax.Precision.DEFAULT) + bc_ref[...]


def _sc_segment_sum(hsplit, src2, dst2, n_nodes, groups_per_sub, acc_rows):
    h = hsplit.shape[2]
    mesh = plsc.VectorSubcoreMesh(core_axis_name="c", subcore_axis_name="s")
    zper = acc_rows // _NSUB
    half = groups_per_sub // _CHUNKS

    @functools.partial(
        pl.kernel,
        out_type=jax.ShapeDtypeStruct((_NCORE, acc_rows, h), jnp.float32),
        mesh=mesh,
        scratch_types=(
            [pltpu.VMEM((half, _GROUP), jnp.int32)] * 4
            + [pltpu.VMEM((_GROUP, h), jnp.float32)] * _NBUF
            + [pltpu.VMEM_SHARED((acc_rows, h), jnp.float32)]
            + [pltpu.SemaphoreType.DMA] * (_NBUF + 2)
        ),
    )
    def k(h_hbm, s_hbm, d_hbm, o_hbm, *rest):
        slabs = ((rest[0], rest[1]), (rest[2], rest[3]))
        bufs = rest[4:4 + _NBUF]
        acc = rest[4 + _NBUF]
        sems = rest[5 + _NBUF:5 + 2 * _NBUF]
        lsem0, lsem1 = rest[5 + 2 * _NBUF], rest[6 + 2 * _NBUF]
        c = lax.axis_index("c")
        s = lax.axis_index("s")
        rows = bufs[0]
        hc = h_hbm.at[c]

        @pl.loop(0, _GROUP)
        def _(r):
            @pl.loop(0, h, step=_LANES)
            def _(i):
                rows.at[pl.ds(r, 1), pl.ds(i, _LANES)][...] = (
                    jnp.zeros((1, _LANES), jnp.float32))

        zfull = (zper // _GROUP) * _GROUP

        @pl.loop(0, zfull, step=_GROUP)
        def _(r0):
            pltpu.sync_copy(rows, acc.at[pl.ds(s * zper + r0, _GROUP)])
        if zfull != zper:
            pltpu.sync_copy(rows.at[pl.ds(0, zper - zfull)],
                            acc.at[pl.ds(s * zper + zfull, zper - zfull)])

        plsc.subcore_barrier()

        base = s * groups_per_sub
        pltpu.sync_copy(s_hbm.at[pl.ds(base, half)], slabs[0][0])
        pltpu.sync_copy(d_hbm.at[pl.ds(base, half)], slabs[0][1])

        for b in range(_NBUF):
            pltpu.async_copy(hc.at[slabs[0][0].at[b]], bufs[b], sems[b])

        for ch in range(_CHUNKS):
            sA, dA = slabs[ch % 2]
            last = ch + 1 == _CHUNKS
            if not last:
                sB, dB = slabs[(ch + 1) % 2]
                nb = base + (ch + 1) * half
                pltpu.async_copy(s_hbm.at[pl.ds(nb, half)], sB, lsem0)
                pltpu.async_copy(d_hbm.at[pl.ds(nb, half)], dB, lsem1)

            @pl.loop(0, half - _NBUF, step=_NBUF)
            def _(j, sA=sA, dA=dA):
                for b in range(_NBUF):
                    pltpu.make_async_copy(hc.at[sA.at[j + b]], bufs[b],
                                          sems[b]).wait()
                    pltpu.sync_copy(bufs[b], acc.at[dA.at[j + b]], add=True)
                    pltpu.async_copy(hc.at[sA.at[j + b + _NBUF]], bufs[b],
                                     sems[b])

            if not last:
                pltpu.make_async_copy(s_hbm.at[pl.ds(nb, half)], sB,
                                      lsem0).wait()
                pltpu.make_async_copy(d_hbm.at[pl.ds(nb, half)], dB,
                                      lsem1).wait()

            for b in range(_NBUF):
                pltpu.make_async_copy(hc.at[sA.at[half - _NBUF + b]], bufs[b],
                                      sems[b]).wait()
                pltpu.sync_copy(bufs[b], acc.at[dA.at[half - _NBUF + b]],
                                add=True)
                if not last:
                    pltpu.async_copy(hc.at[sB.at[b]], bufs[b], sems[b])

        plsc.subcore_barrier()

        pltpu.sync_copy(acc.at[pl.ds(s * zper, zper)],
                        o_hbm.at[c, pl.ds(s * zper, zper)])

    return k(hsplit, src2, dst2)


def kernel(feat, edge_index, W_aggr, b_aggr, W_self, b_self, W_comb, b_comb):
    n, d = feat.shape
    e = edge_index.shape[1]
    h = d // 2
    m_blk = 2000
    grid = n // m_blk

    hsplit = pl.pallas_call(
        _linear_split_kernel,
        grid=(grid,),
        in_specs=[pl.BlockSpec((m_blk, d), lambda i: (i, 0)),
                  pl.BlockSpec((d, d), lambda i: (0, 0)),
                  pl.BlockSpec((1, d), lambda i: (0, 0))],
        out_specs=pl.BlockSpec((2, m_blk, h), lambda i: (0, i, 0)),
        out_shape=jax.ShapeDtypeStruct((2, n, h), jnp.float32),
    )(feat, W_aggr, b_aggr.reshape(1, d))

    unit = _GROUP * _NSUB * 8
    e_pad = ((e + unit - 1) // unit) * unit
    src = edge_index[0].astype(jnp.int32)
    dst = edge_index[1].astype(jnp.int32)
    if e_pad != e:
        src = jnp.concatenate([src, jnp.zeros((e_pad - e,), jnp.int32)])
        dst = jnp.concatenate([dst, jnp.full((e_pad - e,), n, jnp.int32)])
    src2 = src.reshape(e_pad // _GROUP, _GROUP)
    dst2 = dst.reshape(e_pad // _GROUP, _GROUP)
    groups_per_sub = e_pad // (_GROUP * _NSUB)

    acc_rows = ((n + (1 if e_pad != e else 0) + 127) // 128) * 128

    aggr = _sc_segment_sum(hsplit, src2, dst2, n, groups_per_sub, acc_rows)

    out = pl.pallas_call(
        _combine_kernel,
        grid=(grid,),
        in_specs=[pl.BlockSpec((m_blk, d), lambda i: (i, 0)),
                  pl.BlockSpec((2, m_blk, h), lambda i: (0, i, 0)),
                  pl.BlockSpec((d, d), lambda i: (0, 0)),
                  pl.BlockSpec((1, d), lambda i: (0, 0)),
                  pl.BlockSpec((d, d), lambda i: (0, 0)),
                  pl.BlockSpec((1, d), lambda i: (0, 0))],
        out_specs=pl.BlockSpec((m_blk, d), lambda i: (i, 0)),
        out_shape=jax.ShapeDtypeStruct((n, d), jnp.float32),
    )(feat, aggr, W_self, b_self.reshape(1, d), W_comb, b_comb.reshape(1, d))
    return out
